# baseline (speedup 1.0000x reference)
import jax
import jax.numpy as jnp
from jax import lax
from jax.experimental import pallas as pl
from jax.experimental.pallas import tpu as pltpu
from jax.experimental.pallas import tpu_sc as plsc

B = 1024
L = 200
D = 128
NC = 2
NS = 16
NW = NC * NS
ROWS = B * L
BPW = ROWS // NW
CH = 200
NCHUNK = BPW // CH
HALF = CH // 2
SPLIT = 96
NBUF = 3
LANES = 16


def _sc_body(idx_hbm, token_hbm, pos_hbm, out_hbm,
             idx_v, pos_v, buf0, buf1, buf2, gsem, ssem):
    c_ax = lax.axis_index("c")
    s_ax = lax.axis_index("s")
    wid = s_ax * NC + c_ax
    bufs = (buf0, buf1, buf2)

    pltpu.sync_copy(idx_hbm.at[wid], idx_v)
    pltpu.sync_copy(pos_hbm, pos_v)

    def gather_half(ci, b, h, issue):
        f = pltpu.async_copy if issue else pltpu.make_async_copy
        return f(token_hbm.at[idx_v.at[2 * ci + h]],
                 bufs[b].at[pl.ds(h * HALF, HALF)], gsem.at[b, h])

    def store_piece(ci, b, lo, n, issue):
        base = wid * BPW + ci * CH
        return (pltpu.async_copy if issue else pltpu.make_async_copy)(
            bufs[b].at[pl.ds(lo, n)], out_hbm.at[pl.ds(base + lo, n)],
            ssem.at[b])

    def store_wait(ci, b):
        store_piece(ci, b, 0, SPLIT, False).wait()
        store_piece(ci, b, SPLIT, CH - SPLIT, False).wait()

    def add_rows(buf, lo, hi):
        @plsc.parallel_loop(lo, hi, unroll=4)
        def _(r):
            for j in range(D // LANES):
                sl = pl.ds(j * LANES, LANES)
                buf[r, sl] = buf[r, sl] + pos_v[r, sl]

    def do_chunk(ci, b):
        nb = (b + 1) % NBUF

        @pl.when(jnp.logical_and(ci >= 2, ci + 1 < NCHUNK))
        def _():
            store_wait(ci - 2, nb)

        @pl.when(ci + 1 < NCHUNK)
        def _():
            gather_half(ci + 1, nb, 0, True)
            gather_half(ci + 1, nb, 1, True)

        buf = bufs[b]

        gather_half(ci, b, 0, False).wait()
        add_rows(buf, 0, SPLIT)
        store_piece(ci, b, 0, SPLIT, True)

        gather_half(ci, b, 1, False).wait()
        add_rows(buf, SPLIT, CH)
        store_piece(ci, b, SPLIT, CH - SPLIT, True)

    gather_half(0, 0, 0, True)
    gather_half(0, 0, 1, True)

    def group_body(g, carry):
        for b in range(NBUF):
            do_chunk(g * NBUF + b, b)
        return carry

    ngroups = NCHUNK // NBUF
    lax.fori_loop(0, ngroups, group_body, 0)
    for tail in range(ngroups * NBUF, NCHUNK):
        do_chunk(tail, tail % NBUF)

    for ci in range(NCHUNK - NBUF, NCHUNK):
        store_wait(ci, ci % NBUF)


@jax.jit
def _embed(idx, token_table, pos_table):
    mesh = plsc.VectorSubcoreMesh(core_axis_name="c", subcore_axis_name="s")
    f = pl.kernel(
        _sc_body,
        out_type=jax.ShapeDtypeStruct((ROWS, D), jnp.float32),
        mesh=mesh,
        scratch_types=[
            pltpu.VMEM((NCHUNK * 2, HALF), jnp.int32),
            pltpu.VMEM((L, D), jnp.float32),
            pltpu.VMEM((CH, D), jnp.float32),
            pltpu.VMEM((CH, D), jnp.float32),
            pltpu.VMEM((CH, D), jnp.float32),
            pltpu.SemaphoreType.DMA((NBUF, 2)),
            pltpu.SemaphoreType.DMA((NBUF,)),
        ],
    )
    return f(idx, token_table, pos_table)


def kernel(inputs, token_table, pos_table):
    idx = inputs.reshape(NW, NCHUNK * 2, HALF).astype(jnp.int32)
    out = _embed(idx, token_table, pos_table)
    return out.reshape(B, L, D)
